# baseline (speedup 1.0000x reference)
import numpy as np
import jax
from jax.experimental import pallas as pl
from jax.experimental.pallas import tpu as pltpu

_NCHUNK = 8


def kernel(frames):
    C, T, H, W = frames.shape
    S = T // 4
    idx = [int(v) for v in np.linspace(0, T - 1, S).astype(np.int64)]
    step = T // _NCHUNK

    def body(x_ref, slow_ref, fast_ref, sem):
        copies = []
        for k in range(_NCHUNK):
            copies.append(pltpu.make_async_copy(
                x_ref.at[:, k * step:(k + 1) * step],
                fast_ref.at[:, k * step:(k + 1) * step],
                sem))
        for j, t in enumerate(idx):
            copies.append(pltpu.make_async_copy(
                x_ref.at[:, t:t + 1], slow_ref.at[:, j:j + 1], sem))
        for c in copies:
            c.start()
        for c in copies:
            c.wait()

    slow, fast = pl.pallas_call(
        body,
        in_specs=[pl.BlockSpec(memory_space=pl.ANY)],
        out_specs=[
            pl.BlockSpec(memory_space=pl.ANY),
            pl.BlockSpec(memory_space=pl.ANY),
        ],
        out_shape=[
            jax.ShapeDtypeStruct((C, S, H, W), frames.dtype),
            jax.ShapeDtypeStruct((C, T, H, W), frames.dtype),
        ],
        scratch_shapes=[pltpu.SemaphoreType.DMA],
    )(frames)
    return (slow, fast)


# speedup vs baseline: 11.3069x; 11.3069x over previous
import numpy as np
import jax
from jax.experimental import pallas as pl
from jax.experimental.pallas import tpu as pltpu

_NBUF = 6
_PRE = 3


def kernel(frames):
    C, T, H, W = frames.shape
    S = T // 4
    idx = [int(v) for v in np.linspace(0, T - 1, S).astype(np.int64)]
    slot_of = {t: j for j, t in enumerate(idx)}
    L = 128
    R = (H * W) // L
    x = frames.reshape(C, T, R, L)

    def body(x_ref, slow_ref, fast_ref, buf, in_sem, out_sem):
        def in_copy(t):
            return pltpu.make_async_copy(
                x_ref.at[:, t:t + 1], buf.at[t % _NBUF], in_sem.at[t % _NBUF])

        def out_copies(t):
            cs = [pltpu.make_async_copy(
                buf.at[t % _NBUF], fast_ref.at[:, t:t + 1],
                out_sem.at[t % _NBUF])]
            if t in slot_of:
                j = slot_of[t]
                cs.append(pltpu.make_async_copy(
                    buf.at[t % _NBUF], slow_ref.at[:, j:j + 1],
                    out_sem.at[t % _NBUF]))
            return cs

        for t in range(_PRE):
            in_copy(t).start()
        for t in range(T):
            look = t + _PRE
            if look < T:
                prev = look - _NBUF
                if prev >= 0:
                    for c in out_copies(prev):
                        c.wait()
                in_copy(look).start()
            in_copy(t).wait()
            for c in out_copies(t):
                c.start()
        for t in range(T - _NBUF, T):
            for c in out_copies(t):
                c.wait()

    slow, fast = pl.pallas_call(
        body,
        in_specs=[pl.BlockSpec(memory_space=pl.ANY)],
        out_specs=[
            pl.BlockSpec(memory_space=pl.ANY),
            pl.BlockSpec(memory_space=pl.ANY),
        ],
        out_shape=[
            jax.ShapeDtypeStruct((C, S, R, L), frames.dtype),
            jax.ShapeDtypeStruct((C, T, R, L), frames.dtype),
        ],
        scratch_shapes=[
            pltpu.VMEM((_NBUF, C, 1, R, L), frames.dtype),
            pltpu.SemaphoreType.DMA((_NBUF,)),
            pltpu.SemaphoreType.DMA((_NBUF,)),
        ],
    )(x)
    return (slow.reshape(C, S, H, W), fast.reshape(C, T, H, W))


# speedup vs baseline: 11.7329x; 1.0377x over previous
import numpy as np
import jax
from jax.experimental import pallas as pl
from jax.experimental.pallas import tpu as pltpu

_CHUNK = 16
_NBUF = 4
_PRE = 2


def kernel(frames):
    C, T, H, W = frames.shape
    S = T // 4
    idx = [int(v) for v in np.linspace(0, T - 1, S).astype(np.int64)]
    L = 128
    R = (H * W) // L
    x = frames.reshape(C * T, R, L)
    nrows = C * T
    nchunks = nrows // _CHUNK
    gather = {c * T + t: c * S + j
              for c in range(C) for j, t in enumerate(idx)}
    chunk_gather = [
        [(r - k * _CHUNK, gather[r])
         for r in range(k * _CHUNK, (k + 1) * _CHUNK) if r in gather]
        for k in range(nchunks)
    ]

    def body(x_ref, slow_ref, fast_ref, buf, in_sem, out_sem):
        def in_copy(k):
            return pltpu.make_async_copy(
                x_ref.at[k * _CHUNK:(k + 1) * _CHUNK],
                buf.at[k % _NBUF], in_sem.at[k % _NBUF])

        def out_copies(k):
            cs = [pltpu.make_async_copy(
                buf.at[k % _NBUF], fast_ref.at[k * _CHUNK:(k + 1) * _CHUNK],
                out_sem.at[k % _NBUF])]
            for off, dst in chunk_gather[k]:
                cs.append(pltpu.make_async_copy(
                    buf.at[k % _NBUF, off:off + 1],
                    slow_ref.at[dst:dst + 1], out_sem.at[k % _NBUF]))
            return cs

        for k in range(_PRE):
            in_copy(k).start()
        for k in range(nchunks):
            look = k + _PRE
            if look < nchunks:
                prev = look - _NBUF
                if prev >= 0:
                    for c in out_copies(prev):
                        c.wait()
                in_copy(look).start()
            in_copy(k).wait()
            for c in out_copies(k):
                c.start()
        for k in range(max(0, nchunks - _NBUF), nchunks):
            for c in out_copies(k):
                c.wait()

    slow, fast = pl.pallas_call(
        body,
        in_specs=[pl.BlockSpec(memory_space=pl.ANY)],
        out_specs=[
            pl.BlockSpec(memory_space=pl.ANY),
            pl.BlockSpec(memory_space=pl.ANY),
        ],
        out_shape=[
            jax.ShapeDtypeStruct((C * S, R, L), frames.dtype),
            jax.ShapeDtypeStruct((C * T, R, L), frames.dtype),
        ],
        scratch_shapes=[
            pltpu.VMEM((_NBUF, _CHUNK, R, L), frames.dtype),
            pltpu.SemaphoreType.DMA((_NBUF,)),
            pltpu.SemaphoreType.DMA((_NBUF,)),
        ],
    )(x)
    return (slow.reshape(C, S, H, W), fast.reshape(C, T, H, W))


# speedup vs baseline: 15.0865x; 1.2858x over previous
import numpy as np
import jax
from jax.experimental import pallas as pl


def _gather_body(x_ref, slow_ref):
    slow_ref[...] = x_ref[...]


def kernel(frames):
    C, T, H, W = frames.shape
    S = T // 4
    idx = [int(v) for v in np.linspace(0, T - 1, S).astype(np.int64)]
    L = 128
    R = (H * W) // L
    x = frames.reshape(C, T, R, L)
    idx_arr = np.asarray(idx, dtype=np.int32)

    def in_map(j):
        return (0, j * (T - 1) // (S - 1), 0, 0)

    slow = pl.pallas_call(
        _gather_body,
        grid=(S,),
        in_specs=[pl.BlockSpec((C, 1, R, L), in_map)],
        out_specs=pl.BlockSpec((C, 1, R, L), lambda j: (0, j, 0, 0)),
        out_shape=jax.ShapeDtypeStruct((C, S, R, L), frames.dtype),
    )(x)
    del idx_arr
    return (slow.reshape(C, S, H, W), frames)


# speedup vs baseline: 51.6098x; 3.4209x over previous
import numpy as np
import jax
from jax.experimental import pallas as pl
from jax.experimental.pallas import tpu as pltpu

_CHUNK = 8
_NBUF = 3
_PRE = 2


def kernel(frames):
    C, T, H, W = frames.shape
    S = T // 4
    idx = [int(v) for v in np.linspace(0, T - 1, S).astype(np.int64)]
    nchunks = T // _CHUNK
    chunk_gather = [
        [(t - k * _CHUNK, j) for j, t in enumerate(idx)
         if k * _CHUNK <= t < (k + 1) * _CHUNK]
        for k in range(nchunks)
    ]

    def body(x_ref, slow_ref, fast_ref, buf, in_sem, out_sem):
        def in_copy(k):
            return pltpu.make_async_copy(
                x_ref.at[:, k * _CHUNK:(k + 1) * _CHUNK],
                buf.at[k % _NBUF], in_sem.at[k % _NBUF])

        def out_copies(k):
            cs = [pltpu.make_async_copy(
                buf.at[k % _NBUF],
                fast_ref.at[:, k * _CHUNK:(k + 1) * _CHUNK],
                out_sem.at[k % _NBUF])]
            for off, j in chunk_gather[k]:
                cs.append(pltpu.make_async_copy(
                    buf.at[k % _NBUF, :, off:off + 1],
                    slow_ref.at[:, j:j + 1], out_sem.at[k % _NBUF]))
            return cs

        for k in range(_PRE):
            in_copy(k).start()
        for k in range(nchunks):
            look = k + _PRE
            if look < nchunks:
                prev = look - _NBUF
                if prev >= 0:
                    for c in out_copies(prev):
                        c.wait()
                in_copy(look).start()
            in_copy(k).wait()
            for c in out_copies(k):
                c.start()
        for k in range(max(0, nchunks - _NBUF), nchunks):
            for c in out_copies(k):
                c.wait()

    slow, fast = pl.pallas_call(
        body,
        in_specs=[pl.BlockSpec(memory_space=pl.ANY)],
        out_specs=[
            pl.BlockSpec(memory_space=pl.ANY),
            pl.BlockSpec(memory_space=pl.ANY),
        ],
        out_shape=[
            jax.ShapeDtypeStruct((C, S, H, W), frames.dtype),
            jax.ShapeDtypeStruct((C, T, H, W), frames.dtype),
        ],
        scratch_shapes=[
            pltpu.VMEM((_NBUF, C, _CHUNK, H, W), frames.dtype),
            pltpu.SemaphoreType.DMA((_NBUF,)),
            pltpu.SemaphoreType.DMA((_NBUF,)),
        ],
    )(frames)
    return (slow, fast)


# speedup vs baseline: 54.7233x; 1.0603x over previous
import numpy as np
import jax
from jax.experimental import pallas as pl
from jax.experimental.pallas import tpu as pltpu

_CHUNK = 16
_NBUF = 3
_PRE = 2


def kernel(frames):
    C, T, H, W = frames.shape
    S = T // 4
    idx = [int(v) for v in np.linspace(0, T - 1, S).astype(np.int64)]
    nchunks = T // _CHUNK
    chunk_gather = [
        [(t - k * _CHUNK, j) for j, t in enumerate(idx)
         if k * _CHUNK <= t < (k + 1) * _CHUNK]
        for k in range(nchunks)
    ]

    def body(x_ref, slow_ref, fast_ref, buf, in_sem, out_sem):
        def in_copy(k):
            return pltpu.make_async_copy(
                x_ref.at[:, k * _CHUNK:(k + 1) * _CHUNK],
                buf.at[k % _NBUF], in_sem.at[k % _NBUF])

        def out_copies(k):
            cs = [pltpu.make_async_copy(
                buf.at[k % _NBUF],
                fast_ref.at[:, k * _CHUNK:(k + 1) * _CHUNK],
                out_sem.at[k % _NBUF])]
            for off, j in chunk_gather[k]:
                cs.append(pltpu.make_async_copy(
                    buf.at[k % _NBUF, :, off:off + 1],
                    slow_ref.at[:, j:j + 1], out_sem.at[k % _NBUF]))
            return cs

        for k in range(_PRE):
            in_copy(k).start()
        for k in range(nchunks):
            look = k + _PRE
            if look < nchunks:
                prev = look - _NBUF
                if prev >= 0:
                    for c in out_copies(prev):
                        c.wait()
                in_copy(look).start()
            in_copy(k).wait()
            for c in out_copies(k):
                c.start()
        for k in range(max(0, nchunks - _NBUF), nchunks):
            for c in out_copies(k):
                c.wait()

    slow, fast = pl.pallas_call(
        body,
        in_specs=[pl.BlockSpec(memory_space=pl.ANY)],
        out_specs=[
            pl.BlockSpec(memory_space=pl.ANY),
            pl.BlockSpec(memory_space=pl.ANY),
        ],
        out_shape=[
            jax.ShapeDtypeStruct((C, S, H, W), frames.dtype),
            jax.ShapeDtypeStruct((C, T, H, W), frames.dtype),
        ],
        scratch_shapes=[
            pltpu.VMEM((_NBUF, C, _CHUNK, H, W), frames.dtype),
            pltpu.SemaphoreType.DMA((_NBUF,)),
            pltpu.SemaphoreType.DMA((_NBUF,)),
        ],
    )(frames)
    return (slow, fast)
